# baseline (speedup 1.0000x reference)
import functools

import numpy as np
import jax
import jax.numpy as jnp
from jax import lax
from jax.experimental import pallas as pl
from jax.experimental.pallas import tpu as pltpu
from jax.experimental.pallas import tpu_sc as plsc

_NUM_VERTICES = 6890
_PATH_LEN = 64
_NUM_SAMPLES = 32

_rng_const = np.random.RandomState(42)
_CIRC_NAMES = ['bicep_left', 'calf_left', 'chest', 'hip', 'neck',
               'thigh_left', 'waist', 'wrist_left']
_CIRC_PATHS = {n: _rng_const.randint(0, _NUM_VERTICES, size=_PATH_LEN)
               for n in _CIRC_NAMES}
_LINEAR_NAMES = ['arm_length_left', 'foot_length_left', 'head_height',
                 'inseam', 'shoulder_breadth', 'torso_length']
_PAIRS = {n: tuple(_rng_const.randint(0, _NUM_VERTICES, size=2).tolist())
          for n in _LINEAR_NAMES}
_SORTED_NAMES = sorted(_CIRC_NAMES + _LINEAR_NAMES)

_NSEG = 8 * _NUM_SAMPLES + 6
_PHASES = (0, 6, 4, 2)


def _build_segments():
    si = np.linspace(0.0, _PATH_LEN - 1.0, _NUM_SAMPLES).astype(np.float32)
    i_floor = np.floor(si).astype(np.int32)
    i_ceil = np.ceil(si).astype(np.int32)
    alpha = (si - i_floor.astype(np.float32)).astype(np.float32)

    segments = []
    for n in _CIRC_NAMES:
        path = _CIRC_PATHS[n]

        def pt_terms(k, sign):
            return [(int(path[i_floor[k]]), sign * (1.0 - float(alpha[k]))),
                    (int(path[i_ceil[k]]), sign * float(alpha[k]))]

        for k in range(_NUM_SAMPLES - 1):
            segments.append(pt_terms(k + 1, 1.0) + pt_terms(k, -1.0))
        segments.append(pt_terms(0, 1.0) + pt_terms(_NUM_SAMPLES - 1, -1.0))
    for n in _LINEAR_NAMES:
        i, j = _PAIRS[n]
        segments.append([(i, 1.0), (j, -1.0), (0, 0.0), (0, 0.0)])
    assert len(segments) == _NSEG
    return segments


def _build_tables():
    segments = _build_segments()
    used = sorted({vid for seg in segments for vid, _ in seg})

    phase_rows = []
    for phi in _PHASES:
        rows = sorted({(3 * v + c + phi) >> 3 for v in used for c in range(3)})
        phase_rows.append(rows)
    nr = max(len(r) for r in phase_rows)
    nr = (nr + 7) // 8 * 8

    rows_tab = np.zeros((4, nr), np.int32)
    seg_tab = np.zeros((4, _NSEG, 16), np.int32)
    for p, phi in enumerate(_PHASES):
        rows = phase_rows[p]
        rows_tab[p, :len(rows)] = np.array(rows, np.int32)
        rows_tab[p, len(rows):] = rows[0]
        rowpos = {r: k for k, r in enumerate(rows)}
        for s, seg in enumerate(segments):
            for t, (vid, cf) in enumerate(seg):
                for c in range(3):
                    w = 3 * vid + c + phi
                    seg_tab[p, s, t * 3 + c] = rowpos[w >> 3] * 8 + (w & 7)
                seg_tab[p, s, 12 + t] = np.float32(cf * 1000.0).view(np.int32)
    return nr, rows_tab, seg_tab.reshape(-1)


_NR, _ROWS_TAB, _SEG_TAB = _build_tables()

_NW = 32
_EPR = 16
_CHUNK = 128
_NCHUNK = _EPR * _NR // _CHUNK
_NMEAS = 16


def _sc_body(rounds, verts_hbm, idx_hbm, segtab_hbm, out_hbm,
             idx_v, rows_v, seg_v, norms_v, stage_v, sem):
    wid = lax.axis_index("s") * 2 + lax.axis_index("c")
    rpp = rounds // 4

    pltpu.sync_copy(segtab_hbm, seg_v)

    lane = lax.iota(jnp.int32, 16)
    e_row = lane * _NR
    zero = jnp.zeros((16,), jnp.float32)
    zero_i = jnp.zeros((16,), jnp.int32)

    def round_body(r, carry):
        pltpu.sync_copy(idx_hbm.at[wid * rounds + r], idx_v)

        def fire(j, c3):
            pltpu.async_copy(
                verts_hbm.at[idx_v.at[j]],
                rows_v.at[pl.ds(j * _CHUNK, _CHUNK)], sem)
            return c3

        lax.fori_loop(0, _NCHUNK, fire, 0)
        pltpu.make_async_copy(
            verts_hbm.at[pl.ds(0, _EPR * _NR)], rows_v, sem).wait()

        p = r // rpp
        pbase = p * (_NSEG * 16)

        def seg_body(s, c2):
            tv = seg_v[pl.ds(pbase + s * 16, 16)]
            cfv = plsc.bitcast(tv, jnp.float32)
            d = [zero, zero, zero]
            for t in range(4):
                cf = cfv[12 + t]
                for c in range(3):
                    off = tv[t * 3 + c]
                    rowi = e_row + lax.shift_right_logical(off, 3)
                    coli = zero_i + (off & 7)
                    v = plsc.load_gather(rows_v, [rowi, coli])
                    d[c] = d[c] + cf * v
            sq = d[0] * d[0] + d[1] * d[1] + d[2] * d[2]
            sqc = jnp.maximum(sq, jnp.float32(1e-30))
            ib = plsc.bitcast(sqc, jnp.int32)
            ib = jnp.int32(0x5F3759DF) - lax.shift_right_logical(ib, jnp.int32(1))
            y = plsc.bitcast(ib, jnp.float32)
            xh = jnp.float32(0.5) * sqc
            y = y * (jnp.float32(1.5) - xh * y * y)
            y = y * (jnp.float32(1.5) - xh * y * y)
            y = y * (jnp.float32(1.5) - xh * y * y)
            norms_v[s] = sq * y
            return c2

        lax.fori_loop(0, _NSEG, seg_body, 0)

        for m, name in enumerate(_SORTED_NAMES):
            if name in _CIRC_NAMES:
                ci = _CIRC_NAMES.index(name)
                acc = norms_v[32 * ci]
                for k in range(1, 32):
                    acc = acc + norms_v[32 * ci + k]
            else:
                acc = norms_v[256 + _LINEAR_NAMES.index(name)]
            plsc.store_scatter(stage_v, [lane, zero_i + m], acc)

        q0 = wid * (rounds * 4) + (r % rpp) * 16
        pltpu.sync_copy(
            stage_v, out_hbm.at[pl.ds(q0, 16), pl.ds(p * 16, 16)])
        return carry

    lax.fori_loop(0, rounds, round_body, 0)


@functools.partial(jax.jit, static_argnames=("batch", "nverts"))
def _measure_sc(verts2d, batch, nverts):
    rounds = batch // (_NW * _EPR)
    rpp = rounds // 4
    nwords = nverts * 3

    wid = jnp.arange(_NW, dtype=jnp.int32)
    r = jnp.arange(rounds, dtype=jnp.int32)
    p = r // rpp
    j = r % rpp
    i = jnp.arange(_EPR, dtype=jnp.int32)
    b = (wid[:, None, None] * (batch // _NW)
         + p[None, :, None] + 4 * (_EPR * j[None, :, None] + i[None, None, :]))
    base_row = (b * nwords) >> 3
    rows_tab = jnp.asarray(_ROWS_TAB)
    idx = base_row[..., None] + rows_tab[p][None, :, None, :]
    idx3d = idx.reshape(_NW * rounds, _NCHUNK, _CHUNK)

    mesh = plsc.VectorSubcoreMesh(core_axis_name="c", subcore_axis_name="s")
    run = pl.kernel(
        functools.partial(_sc_body, rounds),
        out_type=jax.ShapeDtypeStruct((batch // 4, 64), jnp.float32),
        mesh=mesh,
        compiler_params=pltpu.CompilerParams(
            needs_layout_passes=False, use_tc_tiling_on_sc=False),
        scratch_types=[
            pltpu.VMEM((_NCHUNK, _CHUNK), jnp.int32),
            pltpu.VMEM((_EPR * _NR, 8), jnp.float32),
            pltpu.VMEM((4 * _NSEG * 16,), jnp.int32),
            pltpu.VMEM((_NSEG, 16), jnp.float32),
            pltpu.VMEM((16, 16), jnp.float32),
            pltpu.SemaphoreType.DMA,
        ],
    )
    return run(verts2d, idx3d, jnp.asarray(_SEG_TAB))


def kernel(vertices):
    batch, nverts, _ = vertices.shape
    out = _measure_sc(vertices.reshape(batch * nverts * 3 // 8, 8),
                      batch, nverts)
    return out.reshape(batch, _NMEAS)[:, :14].T


# speedup vs baseline: 1416.4626x; 1416.4626x over previous
import functools

import numpy as np
import jax
import jax.numpy as jnp
from jax import lax
from jax.experimental import pallas as pl
from jax.experimental.pallas import tpu as pltpu
from jax.experimental.pallas import tpu_sc as plsc

_NUM_VERTICES = 6890
_PATH_LEN = 64
_NUM_SAMPLES = 32

_rng_const = np.random.RandomState(42)
_CIRC_NAMES = ['bicep_left', 'calf_left', 'chest', 'hip', 'neck',
               'thigh_left', 'waist', 'wrist_left']
_CIRC_PATHS = {n: _rng_const.randint(0, _NUM_VERTICES, size=_PATH_LEN)
               for n in _CIRC_NAMES}
_LINEAR_NAMES = ['arm_length_left', 'foot_length_left', 'head_height',
                 'inseam', 'shoulder_breadth', 'torso_length']
_PAIRS = {n: tuple(_rng_const.randint(0, _NUM_VERTICES, size=2).tolist())
          for n in _LINEAR_NAMES}
_SORTED_NAMES = sorted(_CIRC_NAMES + _LINEAR_NAMES)

_NSEG = 8 * _NUM_SAMPLES + 6
_PHASE_SEGS = ((0, 128), (128, 262))
_NP = 256


def _build_segments():
    si = np.linspace(0.0, _PATH_LEN - 1.0, _NUM_SAMPLES).astype(np.float32)
    i_floor = np.floor(si).astype(np.int32)
    i_ceil = np.ceil(si).astype(np.int32)
    alpha = (si - i_floor.astype(np.float32)).astype(np.float32)

    segments = []
    for n in _CIRC_NAMES:
        path = _CIRC_PATHS[n]

        def pt_terms(k, sign):
            return [(int(path[i_floor[k]]), sign * (1.0 - float(alpha[k]))),
                    (int(path[i_ceil[k]]), sign * float(alpha[k]))]

        for k in range(_NUM_SAMPLES - 1):
            segments.append(pt_terms(k + 1, 1.0) + pt_terms(k, -1.0))
        segments.append(pt_terms(0, 1.0) + pt_terms(_NUM_SAMPLES - 1, -1.0))
    for n in _LINEAR_NAMES:
        i, j = _PAIRS[n]
        segments.append([(i, 1.0), (j, -1.0), (0, 0.0), (0, 0.0)])
    assert len(segments) == _NSEG
    return segments


def _build_tables():
    segments = _build_segments()
    vidx_tab = np.zeros((2, _NP), np.int32)
    seg_tab = np.zeros((_NSEG, 16), np.int32)
    for p, (s0, s1) in enumerate(_PHASE_SEGS):
        used = sorted({vid for seg in segments[s0:s1] for vid, _ in seg})
        assert len(used) <= _NP, (p, len(used))
        vidx_tab[p, :len(used)] = np.array(used, np.int32)
        vidx_tab[p, len(used):] = used[0]
        pos_of = {v: k for k, v in enumerate(used)}
        for s in range(s0, s1):
            for t, (vid, cf) in enumerate(segments[s]):
                for c in range(3):
                    seg_tab[s, t * 3 + c] = c * _NP + pos_of[vid]
                seg_tab[s, 12 + t] = np.float32(cf * 1000.0).view(np.int32)
    return vidx_tab.reshape(2, 2, 128), seg_tab.reshape(-1)


_VIDX_TAB, _SEG_TAB = _build_tables()

_CIRC_M = [_SORTED_NAMES.index(n) for n in _CIRC_NAMES]
_LIN_M = [_SORTED_NAMES.index(n) for n in _LINEAR_NAMES]

_NW = 32


def _newton_norm(sq):
    sqc = jnp.maximum(sq, jnp.float32(1e-30))
    ib = plsc.bitcast(sqc, jnp.int32)
    ib = jnp.int32(0x5F3759DF) - lax.shift_right_logical(ib, jnp.int32(1))
    y = plsc.bitcast(ib, jnp.float32)
    xh = jnp.float32(0.5) * sqc
    y = y * (jnp.float32(1.5) - xh * y * y)
    y = y * (jnp.float32(1.5) - xh * y * y)
    y = y * (jnp.float32(1.5) - xh * y * y)
    return sq * y


def _sc_body(verts_hbm, vidx_hbm, segtab_hbm, out_hbm,
             idx_v, rows_v, seg_v, stage_v, sem):
    wid = lax.axis_index("s") * 2 + lax.axis_index("c")
    b0 = wid * 128

    pltpu.sync_copy(segtab_hbm, seg_v)
    zero = jnp.zeros((16,), jnp.float32)

    for p, (s0, s1) in enumerate(_PHASE_SEGS):
        pltpu.sync_copy(vidx_hbm.at[p], idx_v)
        for c in range(3):
            for k in range(_NP // 128):
                pltpu.async_copy(
                    verts_hbm.at[c].at[idx_v.at[k], pl.ds(b0, 128)],
                    rows_v.at[pl.ds(c * _NP + k * 128, 128)], sem)
        pltpu.make_async_copy(
            verts_hbm.at[0, pl.ds(0, 3 * _NP), pl.ds(0, 128)],
            rows_v, sem).wait()

        def seg_norm(s, g16):
            tv = seg_v[pl.ds(s * 16, 16)]
            cfv = plsc.bitcast(tv, jnp.float32)
            d = [zero, zero, zero]
            for t in range(4):
                cf = cfv[12 + t]
                for c in range(3):
                    val = rows_v[tv[t * 3 + c], pl.ds(g16, 16)]
                    d[c] = d[c] + cf * val
            return _newton_norm(d[0] * d[0] + d[1] * d[1] + d[2] * d[2])

        for cr in range(4):
            ci = 4 * p + cr

            def one_group(g, c4, ci=ci):
                g16 = g * 16
                acc = lax.fori_loop(
                    32 * ci, 32 * ci + 32,
                    lambda s, a: a + seg_norm(s, g16), zero)
                stage_v[_CIRC_M[ci], pl.ds(g16, 16)] = acc
                return c4

            lax.fori_loop(0, 8, one_group, 0)
        if p == 1:
            for li in range(6):

                def lin_group(g, c4, li=li):
                    g16 = g * 16
                    stage_v[_LIN_M[li], pl.ds(g16, 16)] = \
                        seg_norm(256 + li, g16)
                    return c4

                lax.fori_loop(0, 8, lin_group, 0)

    pltpu.sync_copy(stage_v, out_hbm.at[:, pl.ds(b0, 128)])


@functools.partial(jax.jit, static_argnames=("batch",))
def _measure_sc(verts_t, batch):
    mesh = plsc.VectorSubcoreMesh(core_axis_name="c", subcore_axis_name="s")
    run = pl.kernel(
        _sc_body,
        out_type=jax.ShapeDtypeStruct((16, batch), jnp.float32),
        mesh=mesh,
        compiler_params=pltpu.CompilerParams(needs_layout_passes=False),
        scratch_types=[
            pltpu.VMEM((2, 128), jnp.int32),
            pltpu.VMEM((3 * _NP, 128), jnp.float32),
            pltpu.VMEM((_NSEG * 16,), jnp.int32),
            pltpu.VMEM((16, 128), jnp.float32),
            pltpu.SemaphoreType.DMA,
        ],
    )
    return run(verts_t, jnp.asarray(_VIDX_TAB), jnp.asarray(_SEG_TAB))


def kernel(vertices):
    batch = vertices.shape[0]
    out = _measure_sc(jnp.transpose(vertices, (2, 1, 0)), batch)
    return out[:14]
